# Initial kernel scaffold; baseline (speedup 1.0000x reference)
import jax
import jax.numpy as jnp
from jax.experimental import pallas as pl


def kernel(src_word, src_pos, emb):
    raise NotImplementedError("write your pallas kernel here")


# speedup vs baseline: 1.7151x; 1.7151x over previous
import functools

import jax
import jax.numpy as jnp
from jax import lax
from jax.experimental import pallas as pl
from jax.experimental.pallas import tpu as pltpu
from jax.experimental.pallas import tpu_sc as plsc

D = 64
R = 4096 * 200
NW = 32
ROWS_PER_W = R // NW
CHUNK = 512
NCHUNK = ROWS_PER_W // CHUNK
IDXG = 128
NIDXG = CHUNK // IDXG
SCALE = float(D) ** 0.5

_mesh = plsc.VectorSubcoreMesh(core_axis_name="c", subcore_axis_name="s")


@functools.partial(
    pl.kernel,
    mesh=_mesh,
    out_type=jax.ShapeDtypeStruct((R, D), jnp.float32),
    compiler_params=pltpu.CompilerParams(use_tc_tiling_on_sc=False),
    scratch_types=[
        pltpu.VMEM((CHUNK, D), jnp.float32),
        pltpu.VMEM((CHUNK, D), jnp.float32),
        pltpu.VMEM((CHUNK,), jnp.int32),
        pltpu.SemaphoreType.DMA,
    ],
)
def _sc_kernel(src_hbm, pos_hbm, emb_hbm, out_hbm, a_v, g_v, idx_v, sem):
    wid = lax.axis_index("s") * 2 + lax.axis_index("c")
    base = wid * ROWS_PER_W

    def chunk_body(ci, carry):
        row0 = base + ci * CHUNK
        pltpu.sync_copy(src_hbm.at[pl.ds(row0, CHUNK)], a_v)
        pltpu.sync_copy(pos_hbm.at[pl.ds(row0, CHUNK)], idx_v)
        cps = [
            pltpu.async_copy(
                emb_hbm.at[idx_v.at[pl.ds(j * IDXG, IDXG)]],
                g_v.at[pl.ds(j * IDXG, IDXG)],
                sem,
            )
            for j in range(NIDXG)
        ]
        for cp in cps:
            cp.wait()

        def row_body(r, c2):
            for j in range(D // 16):
                s = pl.ds(j * 16, 16)
                a_v[r, s] = a_v[r, s] * SCALE + g_v[r, s]
            return c2

        lax.fori_loop(0, CHUNK, row_body, 0, unroll=2)
        pltpu.sync_copy(a_v, out_hbm.at[pl.ds(row0, CHUNK)])
        return carry

    lax.fori_loop(0, NCHUNK, chunk_body, 0)


def kernel(src_word, src_pos, emb):
    src = src_word.reshape(R, D).astype(jnp.float32)
    pos = src_pos.reshape(R).astype(jnp.int32)
    out = _sc_kernel(src, pos, emb.astype(jnp.float32))
    return out.reshape(src_word.shape)


# speedup vs baseline: 2.3704x; 1.3820x over previous
import functools

import jax
import jax.numpy as jnp
from jax import lax
from jax.experimental import pallas as pl
from jax.experimental.pallas import tpu as pltpu
from jax.experimental.pallas import tpu_sc as plsc

D = 64
SRC_MAX_LEN = 200
R = 4096 * 200
NW = 32
ROWS_PER_W = R // NW
CHUNK = 512
NCHUNK = ROWS_PER_W // CHUNK
IDXG = 128
NIDXG = CHUNK // IDXG
SCALE = float(D) ** 0.5

_mesh = plsc.VectorSubcoreMesh(core_axis_name="c", subcore_axis_name="s")


@functools.partial(
    pl.kernel,
    mesh=_mesh,
    out_type=jax.ShapeDtypeStruct((R, D), jnp.float32),
    compiler_params=pltpu.CompilerParams(use_tc_tiling_on_sc=False),
    scratch_types=[
        pltpu.VMEM((CHUNK, D), jnp.float32),
        pltpu.VMEM((SRC_MAX_LEN, D), jnp.float32),
        pltpu.VMEM_SHARED((SRC_MAX_LEN, D), jnp.float32),
        pltpu.VMEM((CHUNK,), jnp.int32),
        pltpu.SemaphoreType.DMA,
    ],
)
def _sc_kernel(src_hbm, pos_hbm, emb_hbm, out_hbm, a_v, stage_v, table_s,
               idx_v, sem):
    wid = lax.axis_index("s") * 2 + lax.axis_index("c")
    base = wid * ROWS_PER_W

    @pl.when(lax.axis_index("s") == 0)
    def _stage_table():
        pltpu.sync_copy(emb_hbm, stage_v)

        def scale_body(r, carry):
            for j in range(D // 16):
                s = pl.ds(j * 16, 16)
                stage_v[r, s] = stage_v[r, s] * (1.0 / SCALE)
            return carry

        lax.fori_loop(0, SRC_MAX_LEN, scale_body, 0)
        pltpu.sync_copy(stage_v, table_s)

    plsc.subcore_barrier()

    def chunk_body(ci, carry):
        row0 = base + ci * CHUNK
        pltpu.sync_copy(src_hbm.at[pl.ds(row0, CHUNK)], a_v)
        pltpu.sync_copy(pos_hbm.at[pl.ds(row0, CHUNK)], idx_v)
        cps = [
            pltpu.async_copy(
                table_s.at[idx_v.at[pl.ds(j * IDXG, IDXG)]],
                a_v.at[pl.ds(j * IDXG, IDXG)],
                sem,
                add=True,
            )
            for j in range(NIDXG)
        ]
        for cp in cps:
            cp.wait()

        def row_body(r, c2):
            for j in range(D // 16):
                s = pl.ds(j * 16, 16)
                a_v[r, s] = a_v[r, s] * SCALE
            return c2

        lax.fori_loop(0, CHUNK, row_body, 0, unroll=2)
        pltpu.sync_copy(a_v, out_hbm.at[pl.ds(row0, CHUNK)])
        return carry

    lax.fori_loop(0, NCHUNK, chunk_body, 0)


def kernel(src_word, src_pos, emb):
    src = src_word.reshape(R, D).astype(jnp.float32)
    pos = src_pos.reshape(R).astype(jnp.int32)
    out = _sc_kernel(src, pos, emb.astype(jnp.float32))
    return out.reshape(src_word.shape)
